# Initial kernel scaffold; baseline (speedup 1.0000x reference)
import jax
import jax.numpy as jnp
from jax.experimental import pallas as pl


def kernel(input_word, context_word, W_in, W_out, word_frequency):
    raise NotImplementedError("write your pallas kernel here")


# speedup vs baseline: 8.2823x; 8.2823x over previous
import functools

import jax
import jax.numpy as jnp
from jax import lax
from jax.experimental import pallas as pl
from jax.experimental.pallas import tpu as pltpu
from jax.experimental.pallas import tpu_sc as plsc

B = 16384
D = 64
K = 20
WF = 128
NC = 2
NS = 16
NW = NC * NS
BPW = B // NW
SPW = BPW * K
KP = 32


def _cdf_body(wf_ref, out_ref):
    wf = wf_ref[...]
    logw = jnp.log(jnp.maximum(wf, 1e-30))
    p = jnp.where(wf > 0, jnp.exp(0.75 * logw), 0.0)
    r = lax.broadcasted_iota(jnp.int32, (WF, WF), 0)
    c = lax.broadcasted_iota(jnp.int32, (WF, WF), 1)
    tri = (r <= c).astype(jnp.float32)
    csum = lax.dot_general(p, tri, (((1,), (0,)), ((), ())),
                           preferred_element_type=jnp.float32)
    total = jnp.sum(p, axis=1, keepdims=True)
    out_ref[...] = csum / total


def _make_cdf(word_frequency):
    wf8 = jnp.broadcast_to(word_frequency.reshape(1, WF), (8, WF))
    out = pl.pallas_call(
        _cdf_body,
        out_shape=jax.ShapeDtypeStruct((8, WF), jnp.float32),
    )(wf8)
    return out[0]


def _sc_body(iw_hbm, cw_hbm, win_hbm, wout_hbm, cdf_hbm,
             pos_hbm, scores_hbm,
             iw_idx, cw_idx, iv_rows, ov_rows, wout_l, idx_flat,
             scores_v, cdf_v, sem_g):
    wid = lax.axis_index("s") * NC + lax.axis_index("c")
    base = wid * BPW

    pltpu.sync_copy(cdf_hbm, cdf_v)
    pltpu.sync_copy(wout_hbm.at[pl.ds(0, WF)], wout_l)
    pltpu.sync_copy(iw_hbm.at[pl.ds(base, BPW)], iw_idx)
    pltpu.sync_copy(cw_hbm.at[pl.ds(base, BPW)], cw_idx)

    descs = []
    for j in range(BPW // 128):
        sl = pl.ds(j * 128, 128)
        descs.append(pltpu.async_copy(
            win_hbm.at[iw_idx.at[sl]], iv_rows.at[sl], sem_g))
        descs.append(pltpu.async_copy(
            wout_hbm.at[cw_idx.at[sl]], ov_rows.at[sl], sem_g))

    base_samp = wid * SPW

    def samp_body(v, carry):
        lanei = lax.iota(jnp.int32, 16)
        g = (base_samp + v * 16) + lanei
        h = g * jnp.int32(-1640531527)
        h = h ^ lax.shift_right_logical(h, 16)
        h = h * jnp.int32(-2048144789)
        h = h ^ lax.shift_right_logical(h, 13)
        h = h * jnp.int32(-1028477387)
        h = h ^ lax.shift_right_logical(h, 16)
        ub = lax.shift_right_logical(h, 8)
        u = ub.astype(jnp.float32) * jnp.float32(1.0 / 16777216.0)
        p = jnp.zeros((16,), jnp.int32)
        for s in (64, 32, 16, 8, 4, 2, 1):
            t = p + s
            cv = plsc.load_gather(cdf_v, [t - 1])
            p = jnp.where(u >= cv, t, p)
        idx_flat[pl.ds(v * 16, 16)] = p
        return carry

    lax.fori_loop(0, SPW // 16, samp_body, 0)

    for dsc in descs:
        dsc.wait()

    def dot_body(b, carry):
        s0, s1, s2, s3 = (pl.ds(0, 16), pl.ds(16, 16),
                          pl.ds(32, 16), pl.ds(48, 16))
        iv0 = iv_rows[b, s0]
        iv1 = iv_rows[b, s1]
        iv2 = iv_rows[b, s2]
        iv3 = iv_rows[b, s3]
        ov_rows[b, s0] = ov_rows[b, s0] * iv0
        ov_rows[b, s1] = ov_rows[b, s1] * iv1
        ov_rows[b, s2] = ov_rows[b, s2] * iv2
        ov_rows[b, s3] = ov_rows[b, s3] * iv3
        ja = idx_flat[pl.ds(b * K, 16)]
        jb = idx_flat[pl.ds(b * K + 16, 16)]
        lanei = lax.iota(jnp.int32, 16)
        res_a = jnp.zeros((16,), jnp.float32)
        res_b = jnp.zeros((16,), jnp.float32)
        for kk in range(K):
            j = ja[kk] if kk < 16 else jb[kk - 16]
            acc = wout_l[j, s0] * iv0
            acc = acc + wout_l[j, s1] * iv1
            acc = acc + wout_l[j, s2] * iv2
            acc = acc + wout_l[j, s3] * iv3
            s = jnp.sum(acc)
            if kk < 16:
                res_a = jnp.where(lanei == kk, s, res_a)
            else:
                res_b = jnp.where(lanei == (kk - 16), s, res_b)
        scores_v[b, s0] = res_a
        scores_v[b, s1] = res_b
        return carry

    lax.fori_loop(0, BPW, dot_body, 0)

    pltpu.sync_copy(ov_rows, pos_hbm.at[pl.ds(base, BPW)])
    pltpu.sync_copy(scores_v, scores_hbm.at[wid])


_sc_call = pl.kernel(
    _sc_body,
    out_type=[jax.ShapeDtypeStruct((B, D), jnp.float32),
              jax.ShapeDtypeStruct((NW, BPW, KP), jnp.float32)],
    mesh=plsc.VectorSubcoreMesh(core_axis_name="c", subcore_axis_name="s",
                                num_cores=NC, num_subcores=NS),
    compiler_params=pltpu.CompilerParams(needs_layout_passes=False,
                                         use_tc_tiling_on_sc=False),
    scratch_types=[
        pltpu.VMEM((BPW,), jnp.int32),
        pltpu.VMEM((BPW,), jnp.int32),
        pltpu.VMEM((BPW, D), jnp.float32),
        pltpu.VMEM((BPW, D), jnp.float32),
        pltpu.VMEM((WF, D), jnp.float32),
        pltpu.VMEM((SPW + 16,), jnp.int32),
        pltpu.VMEM((BPW, KP), jnp.float32),
        pltpu.VMEM((WF,), jnp.float32),
        pltpu.SemaphoreType.DMA,
    ],
)


def _reduce_body(prod_ref, sc_ref, out_ref):
    prod = prod_ref[...]
    s = sc_ref[...]
    pos_total = jnp.sum(jnp.mean(jax.nn.log_sigmoid(prod), axis=1))
    col = lax.broadcasted_iota(jnp.int32, s.shape, 1)
    neg_ls = jnp.where(col < K, jax.nn.log_sigmoid(-s), 0.0)
    neg_total = jnp.sum(neg_ls)
    val = -(pos_total + neg_total) / jnp.float32(B)
    out_ref[...] = jnp.full((8, 128), val, jnp.float32)


def _reduce(pos_prod, scores):
    out = pl.pallas_call(
        _reduce_body,
        out_shape=jax.ShapeDtypeStruct((8, 128), jnp.float32),
    )(pos_prod, scores.reshape(NW * BPW, KP))
    return out[0, 0]


def kernel(input_word, context_word, W_in, W_out, word_frequency):
    cdf = _make_cdf(word_frequency)
    pos_prod, scores = _sc_call(input_word, context_word, W_in, W_out, cdf)
    return _reduce(pos_prod, scores)


# speedup vs baseline: 10.0790x; 1.2169x over previous
import functools

import jax
import jax.numpy as jnp
from jax import lax
from jax.experimental import pallas as pl
from jax.experimental.pallas import tpu as pltpu
from jax.experimental.pallas import tpu_sc as plsc

B = 16384
D = 64
K = 20
WF = 128
NC = 2
NS = 16
NW = NC * NS
BPW = B // NW
SPW = BPW * K
KP = 32


def _cdf_body(wf_ref, out_ref):
    wf = wf_ref[...]
    logw = jnp.log(jnp.maximum(wf, 1e-30))
    p = jnp.where(wf > 0, jnp.exp(0.75 * logw), 0.0)
    r = lax.broadcasted_iota(jnp.int32, (WF, WF), 0)
    c = lax.broadcasted_iota(jnp.int32, (WF, WF), 1)
    tri = (r <= c).astype(jnp.float32)
    csum = lax.dot_general(p, tri, (((1,), (0,)), ((), ())),
                           preferred_element_type=jnp.float32)
    total = jnp.sum(p, axis=1, keepdims=True)
    out_ref[...] = csum / total


def _make_cdf(word_frequency):
    wf8 = jnp.broadcast_to(word_frequency.reshape(1, WF), (8, WF))
    out = pl.pallas_call(
        _cdf_body,
        out_shape=jax.ShapeDtypeStruct((8, WF), jnp.float32),
    )(wf8)
    return out[0]


def _sc_body(iw_hbm, cw_hbm, win_hbm, wout_hbm, cdf_hbm,
             pos_hbm, scores_hbm,
             iw_idx, cw_idx, iv_rows, ov_rows, wout_l, idx_flat,
             scores_v, cdf_v, sem_g):
    wid = lax.axis_index("s") * NC + lax.axis_index("c")
    base = wid * BPW

    pltpu.sync_copy(cdf_hbm, cdf_v)
    pltpu.sync_copy(wout_hbm.at[pl.ds(0, WF)], wout_l)
    pltpu.sync_copy(iw_hbm.at[pl.ds(base, BPW)], iw_idx)
    pltpu.sync_copy(cw_hbm.at[pl.ds(base, BPW)], cw_idx)

    descs = []
    for j in range(BPW // 128):
        sl = pl.ds(j * 128, 128)
        descs.append(pltpu.async_copy(
            win_hbm.at[iw_idx.at[sl]], iv_rows.at[sl], sem_g))
        descs.append(pltpu.async_copy(
            wout_hbm.at[cw_idx.at[sl]], ov_rows.at[sl], sem_g))

    base_samp = wid * SPW

    @plsc.parallel_loop(0, SPW // 16, unroll=4)
    def samp_body(v):
        lanei = lax.iota(jnp.int32, 16)
        g = (base_samp + v * 16) + lanei
        h = g * jnp.int32(-1640531527)
        h = h ^ lax.shift_right_logical(h, 16)
        h = h * jnp.int32(-2048144789)
        h = h ^ lax.shift_right_logical(h, 13)
        h = h * jnp.int32(-1028477387)
        h = h ^ lax.shift_right_logical(h, 16)
        ub = lax.shift_right_logical(h, 8)
        u = ub.astype(jnp.float32) * jnp.float32(1.0 / 16777216.0)
        p = jnp.zeros((16,), jnp.int32)
        for s in (64, 32, 16, 8, 4, 2, 1):
            t = p + s
            cv = plsc.load_gather(cdf_v, [t - 1])
            p = jnp.where(u >= cv, t, p)
        idx_flat[pl.ds(v * 16, 16)] = p

    for dsc in descs:
        dsc.wait()

    @plsc.parallel_loop(0, BPW, unroll=2)
    def dot_body(b):
        s0, s1, s2, s3 = (pl.ds(0, 16), pl.ds(16, 16),
                          pl.ds(32, 16), pl.ds(48, 16))
        iv0 = iv_rows[b, s0]
        iv1 = iv_rows[b, s1]
        iv2 = iv_rows[b, s2]
        iv3 = iv_rows[b, s3]
        ov_rows[b, s0] = ov_rows[b, s0] * iv0
        ov_rows[b, s1] = ov_rows[b, s1] * iv1
        ov_rows[b, s2] = ov_rows[b, s2] * iv2
        ov_rows[b, s3] = ov_rows[b, s3] * iv3
        ja = idx_flat[pl.ds(b * K, 16)]
        jb = idx_flat[pl.ds(b * K + 16, 16)]
        lanei = lax.iota(jnp.int32, 16)
        res_a = jnp.zeros((16,), jnp.float32)
        res_b = jnp.zeros((16,), jnp.float32)
        for kk in range(K):
            j = ja[kk] if kk < 16 else jb[kk - 16]
            acc = wout_l[j, s0] * iv0
            acc = acc + wout_l[j, s1] * iv1
            acc = acc + wout_l[j, s2] * iv2
            acc = acc + wout_l[j, s3] * iv3
            s = jnp.sum(acc)
            if kk < 16:
                res_a = jnp.where(lanei == kk, s, res_a)
            else:
                res_b = jnp.where(lanei == (kk - 16), s, res_b)
        scores_v[b, s0] = res_a
        scores_v[b, s1] = res_b

    pltpu.sync_copy(ov_rows, pos_hbm.at[pl.ds(base, BPW)])
    pltpu.sync_copy(scores_v, scores_hbm.at[wid])


_sc_call = pl.kernel(
    _sc_body,
    out_type=[jax.ShapeDtypeStruct((B, D), jnp.float32),
              jax.ShapeDtypeStruct((NW, BPW, KP), jnp.float32)],
    mesh=plsc.VectorSubcoreMesh(core_axis_name="c", subcore_axis_name="s",
                                num_cores=NC, num_subcores=NS),
    compiler_params=pltpu.CompilerParams(needs_layout_passes=False,
                                         use_tc_tiling_on_sc=False),
    scratch_types=[
        pltpu.VMEM((BPW,), jnp.int32),
        pltpu.VMEM((BPW,), jnp.int32),
        pltpu.VMEM((BPW, D), jnp.float32),
        pltpu.VMEM((BPW, D), jnp.float32),
        pltpu.VMEM((WF, D), jnp.float32),
        pltpu.VMEM((SPW + 16,), jnp.int32),
        pltpu.VMEM((BPW, KP), jnp.float32),
        pltpu.VMEM((WF,), jnp.float32),
        pltpu.SemaphoreType.DMA,
    ],
)


def _reduce_body(prod_ref, sc_ref, out_ref):
    prod = prod_ref[...]
    s = sc_ref[...]
    pos_total = jnp.sum(jnp.mean(jax.nn.log_sigmoid(prod), axis=1))
    col = lax.broadcasted_iota(jnp.int32, s.shape, 1)
    neg_ls = jnp.where(col < K, jax.nn.log_sigmoid(-s), 0.0)
    neg_total = jnp.sum(neg_ls)
    val = -(pos_total + neg_total) / jnp.float32(B)
    out_ref[...] = jnp.full((8, 128), val, jnp.float32)


def _reduce(pos_prod, scores):
    out = pl.pallas_call(
        _reduce_body,
        out_shape=jax.ShapeDtypeStruct((8, 128), jnp.float32),
    )(pos_prod, scores.reshape(NW * BPW, KP))
    return out[0, 0]


def kernel(input_word, context_word, W_in, W_out, word_frequency):
    cdf = _make_cdf(word_frequency)
    pos_prod, scores = _sc_call(input_word, context_word, W_in, W_out, cdf)
    return _reduce(pos_prod, scores)


# speedup vs baseline: 10.6804x; 1.0597x over previous
import functools

import jax
import jax.numpy as jnp
from jax import lax
from jax.experimental import pallas as pl
from jax.experimental.pallas import tpu as pltpu
from jax.experimental.pallas import tpu_sc as plsc

B = 16384
D = 64
K = 20
WF = 128
NC = 2
NS = 16
NW = NC * NS
BPW = B // NW
SPW = BPW * K
KP = 32


def _cdf_body(wf_ref, out_ref):
    wf = wf_ref[...]
    logw = jnp.log(jnp.maximum(wf, 1e-30))
    p = jnp.where(wf > 0, jnp.exp(0.75 * logw), 0.0)
    r = lax.broadcasted_iota(jnp.int32, (WF, WF), 0)
    c = lax.broadcasted_iota(jnp.int32, (WF, WF), 1)
    tri = (r <= c).astype(jnp.float32)
    csum = lax.dot_general(p, tri, (((1,), (0,)), ((), ())),
                           preferred_element_type=jnp.float32)
    total = jnp.sum(p, axis=1, keepdims=True)
    out_ref[...] = csum / total


def _make_cdf(word_frequency):
    wf8 = jnp.broadcast_to(word_frequency.reshape(1, WF), (8, WF))
    out = pl.pallas_call(
        _cdf_body,
        out_shape=jax.ShapeDtypeStruct((8, WF), jnp.float32),
    )(wf8)
    return out[0]


def _sc_body(iw_hbm, cw_hbm, win_hbm, wout_hbm, cdf_hbm,
             pos_hbm, scores_hbm,
             iw_idx, cw_idx, iv_rows, ov_rows, wout_l, idx_flat,
             scores_v, pos_buf, cdf_v, sem_g):
    wid = lax.axis_index("s") * NC + lax.axis_index("c")
    base = wid * BPW

    pltpu.sync_copy(cdf_hbm, cdf_v)
    pltpu.sync_copy(wout_hbm.at[pl.ds(0, WF)], wout_l)
    pltpu.sync_copy(iw_hbm.at[pl.ds(base, BPW)], iw_idx)
    pltpu.sync_copy(cw_hbm.at[pl.ds(base, BPW)], cw_idx)

    descs = []
    for j in range(BPW // 128):
        sl = pl.ds(j * 128, 128)
        descs.append(pltpu.async_copy(
            win_hbm.at[iw_idx.at[sl]], iv_rows.at[sl], sem_g))
        descs.append(pltpu.async_copy(
            wout_hbm.at[cw_idx.at[sl]], ov_rows.at[sl], sem_g))

    base_samp = wid * SPW

    @plsc.parallel_loop(0, SPW // 16, unroll=4)
    def samp_body(v):
        lanei = lax.iota(jnp.int32, 16)
        g = (base_samp + v * 16) + lanei
        h = g * jnp.int32(-1640531527)
        h = h ^ lax.shift_right_logical(h, 16)
        h = h * jnp.int32(-2048144789)
        h = h ^ lax.shift_right_logical(h, 13)
        h = h * jnp.int32(-1028477387)
        h = h ^ lax.shift_right_logical(h, 16)
        ub = lax.shift_right_logical(h, 8)
        u = ub.astype(jnp.float32) * jnp.float32(1.0 / 16777216.0)
        p = jnp.zeros((16,), jnp.int32)
        for s in (64, 32, 16, 8, 4, 2, 1):
            t = p + s
            cv = plsc.load_gather(cdf_v, [t - 1])
            p = jnp.where(u >= cv, t, p)
        idx_flat[pl.ds(v * 16, 16)] = p

    for dsc in descs:
        dsc.wait()

    s0, s1, s2, s3 = (pl.ds(0, 16), pl.ds(16, 16),
                      pl.ds(32, 16), pl.ds(48, 16))
    sq = (s0, s1, s2, s3)
    for h in range(2):
        @plsc.parallel_loop(0, BPW // 8, unroll=1)
        def grp_body(g, _h=h):
            b0 = _h * (BPW // 2) + g * 4
            lanei = lax.iota(jnp.int32, 16)
            for i in range(4):
                b = b0 + i
                iv = [iv_rows[b, s] for s in sq]
                ov = [ov_rows[b, s] for s in sq]
                for q in range(4):
                    pos_buf[2 * g + i // 2,
                            pl.ds((i % 2) * 64 + 16 * q, 16)] = ov[q] * iv[q]
                ja = idx_flat[pl.ds(b * K, 16)]
                jb = idx_flat[pl.ds(b * K + 16, 16)]
                res_a = jnp.zeros((16,), jnp.float32)
                res_b = jnp.zeros((16,), jnp.float32)
                for kk in range(K):
                    j = ja[kk] if kk < 16 else jb[kk - 16]
                    acc = wout_l[j, s0] * iv[0]
                    acc = acc + wout_l[j, s1] * iv[1]
                    acc = acc + wout_l[j, s2] * iv[2]
                    acc = acc + wout_l[j, s3] * iv[3]
                    s = jnp.sum(acc)
                    if kk < 16:
                        res_a = jnp.where(lanei == kk, s, res_a)
                    else:
                        res_b = jnp.where(lanei == (kk - 16), s, res_b)
                scores_v[_h * 64 + g, pl.ds(i * 32, 16)] = res_a
                scores_v[_h * 64 + g, pl.ds(i * 32 + 16, 16)] = res_b

        pltpu.sync_copy(pos_buf, pos_hbm.at[wid * 2 + h])

    pltpu.sync_copy(scores_v, scores_hbm.at[wid])


_sc_call = pl.kernel(
    _sc_body,
    out_type=[jax.ShapeDtypeStruct((NW * 2, BPW // 4, 128), jnp.float32),
              jax.ShapeDtypeStruct((NW, BPW // 4, 128), jnp.float32)],
    mesh=plsc.VectorSubcoreMesh(core_axis_name="c", subcore_axis_name="s",
                                num_cores=NC, num_subcores=NS),
    compiler_params=pltpu.CompilerParams(needs_layout_passes=False,
                                         use_tc_tiling_on_sc=False),
    scratch_types=[
        pltpu.VMEM((BPW,), jnp.int32),
        pltpu.VMEM((BPW,), jnp.int32),
        pltpu.VMEM((BPW, D), jnp.float32),
        pltpu.VMEM((BPW, D), jnp.float32),
        pltpu.VMEM((WF, D), jnp.float32),
        pltpu.VMEM((SPW + 16,), jnp.int32),
        pltpu.VMEM((BPW // 4, 128), jnp.float32),
        pltpu.VMEM((BPW // 4, 128), jnp.float32),
        pltpu.VMEM((WF,), jnp.float32),
        pltpu.SemaphoreType.DMA,
    ],
)


def _reduce_body(prod_ref, sc_ref, out_ref):
    prod = prod_ref[...]
    s = sc_ref[...]
    pos_total = jnp.sum(jax.nn.log_sigmoid(prod)) / jnp.float32(D)
    col = lax.broadcasted_iota(jnp.int32, s.shape, 1)
    neg_ls = jnp.where((col & (KP - 1)) < K, jax.nn.log_sigmoid(-s), 0.0)
    neg_total = jnp.sum(neg_ls)
    val = -(pos_total + neg_total) / jnp.float32(B)
    out_ref[...] = jnp.full((8, 128), val, jnp.float32)


def _reduce(pos_prod, scores):
    out = pl.pallas_call(
        _reduce_body,
        out_shape=jax.ShapeDtypeStruct((8, 128), jnp.float32),
    )(pos_prod.reshape(NW * 2 * (BPW // 4), 128),
      scores.reshape(NW * (BPW // 4), 128))
    return out[0, 0]


def kernel(input_word, context_word, W_in, W_out, word_frequency):
    cdf = _make_cdf(word_frequency)
    pos_prod, scores = _sc_call(input_word, context_word, W_in, W_out, cdf)
    return _reduce(pos_prod, scores)
